# Initial kernel scaffold; baseline (speedup 1.0000x reference)
import jax
import jax.numpy as jnp
from jax.experimental import pallas as pl


def kernel(r, v, batch, z, m, emb, W1, b1, W2, b2, E0):
    raise NotImplementedError("write your pallas kernel here")


# speedup vs baseline: 13.2465x; 13.2465x over previous
import dataclasses
import functools

import jax
import jax.numpy as jnp
from jax import lax
from jax.experimental import pallas as pl
from jax.experimental.pallas import tpu as pltpu
from jax.experimental.pallas import tpu_sc as plsc

N = 262144
B = 1024
NZ = 100
DE = 16
H = 64

A = 1024
NB = N // A
NW = 32
CH = N // NW
WIN = 2048
NWIN = CH // WIN

_SC_MESH = dict(core_axis_name="c", subcore_axis_name="s")


def _sc_compiler_params():
    cp = pltpu.CompilerParams()
    if "needs_layout_passes" in pltpu.CompilerParams.__dataclass_fields__:
        cp = dataclasses.replace(cp, needs_layout_passes=False)
    return cp


def _tc_body(rT_ref, vT_ref, z_ref, m_ref, embT_ref, W1eT_ref, W1rT_ref,
             W1r8_ref, W2pT_ref, w2c_ref, b1c_ref, b2_ref,
             ea_ref, gT_ref, p_ref, p_acc):
    i = pl.program_id(0)

    @pl.when(i == 0)
    def _():
        p_acc[...] = jnp.zeros_like(p_acc)

    rT = rT_ref[...]
    vT = vT_ref[...]
    z = z_ref[0]
    m2 = m_ref[0]

    MT = lax.dot_general(W1eT_ref[...], embT_ref[...],
                         (((1,), (0,)), ((), ())),
                         preferred_element_type=jnp.float32)
    idx = jnp.broadcast_to(z, (H, A))
    acc = jnp.take_along_axis(MT, idx, axis=1)
    acc = acc + lax.dot_general(W1rT_ref[...], rT, (((1,), (0,)), ((), ())),
                                preferred_element_type=jnp.float32)
    hpre = acc + b1c_ref[...]
    h = jnp.tanh(hpre)

    w2c = w2c_ref[...]
    hw = h * w2c
    d = w2c - h * hw

    e8 = lax.dot_general(W2pT_ref[...], h, (((1,), (0,)), ((), ())),
                         preferred_element_type=jnp.float32)
    g8 = lax.dot_general(W1r8_ref[...], d, (((1,), (0,)), ((), ())),
                         preferred_element_type=jnp.float32)

    vv = vT * vT
    kin = (vv[0:1] + vv[1:2] + vv[2:3]) * m2
    ea_row = e8[0:1] + b2_ref[...] + 0.5 * kin
    ea_ref[...] = ea_row.reshape(1, 1, A)
    gT_ref[...] = g8[0:3]

    p_acc[...] += vT * m2

    @pl.when(i == NB - 1)
    def _():
        p3 = jnp.sum(p_acc[...], axis=1, keepdims=True)
        p_ref[...] = jnp.broadcast_to(p3, (3, 128))


def _tc_stage(rT, vT, z3, m3, embT, W1eT, W1rT, W1r8, W2pT, w2c, b1c, b2r):
    return pl.pallas_call(
        _tc_body,
        grid=(NB,),
        in_specs=[
            pl.BlockSpec((3, A), lambda i: (0, i)),
            pl.BlockSpec((3, A), lambda i: (0, i)),
            pl.BlockSpec((1, 1, A), lambda i: (i, 0, 0)),
            pl.BlockSpec((1, 1, A), lambda i: (i, 0, 0)),
            pl.BlockSpec((16, 128), lambda i: (0, 0)),
            pl.BlockSpec((64, 16), lambda i: (0, 0)),
            pl.BlockSpec((64, 3), lambda i: (0, 0)),
            pl.BlockSpec((8, 64), lambda i: (0, 0)),
            pl.BlockSpec((8, 64), lambda i: (0, 0)),
            pl.BlockSpec((64, 1), lambda i: (0, 0)),
            pl.BlockSpec((64, 1), lambda i: (0, 0)),
            pl.BlockSpec((1, 1), lambda i: (0, 0)),
        ],
        out_specs=[
            pl.BlockSpec((1, 1, A), lambda i: (i, 0, 0)),
            pl.BlockSpec((3, A), lambda i: (0, i)),
            pl.BlockSpec((3, 128), lambda i: (0, 0)),
        ],
        out_shape=[
            jax.ShapeDtypeStruct((NB, 1, A), jnp.float32),
            jax.ShapeDtypeStruct((3, N), jnp.float32),
            jax.ShapeDtypeStruct((3, 128), jnp.float32),
        ],
        scratch_shapes=[pltpu.VMEM((3, A), jnp.float32)],
    )(rT, vT, z3, m3, embT, W1eT, W1rT, W1r8, W2pT, w2c, b1c, b2r)


@functools.lru_cache(maxsize=None)
def _build_seg_sum():
    @functools.partial(
        pl.kernel,
        mesh=plsc.VectorSubcoreMesh(**_SC_MESH),
        compiler_params=_sc_compiler_params(),
        out_type=jax.ShapeDtypeStruct((NW * B,), jnp.float32),
        scratch_types=[
            pltpu.VMEM((CH,), jnp.float32),
            pltpu.VMEM((CH,), jnp.int32),
            pltpu.VMEM((B,), jnp.float32),
            pltpu.SemaphoreType.DMA,
        ],
    )
    def _seg_sum(ea_hbm, batch_hbm, out_hbm, vals_v, ids_v, acc_v, sem):
        wid = lax.axis_index("c") * 16 + lax.axis_index("s")
        base = wid * CH
        cp1 = pltpu.async_copy(ea_hbm.at[pl.ds(base, CH)], vals_v, sem)
        cp2 = pltpu.async_copy(batch_hbm.at[pl.ds(base, CH)], ids_v, sem)

        zero16 = jnp.zeros((16,), jnp.float32)

        @pl.loop(0, B, step=16)
        def _(k):
            acc_v[pl.ds(k, 16)] = zero16

        cp1.wait()
        cp2.wait()

        iota = lax.iota(jnp.int32, 16)
        m_last = iota == 15
        m_first = iota == 0
        sh_next = jnp.minimum(iota + 1, 15)
        sh_prev = jnp.maximum(iota - 1, 0)

        @pl.loop(0, CH, step=16)
        def _(c):
            ids = ids_v[pl.ds(c, 16)]
            vals = vals_v[pl.ds(c, 16)]
            ids_n = plsc.load_gather(ids_v, [c + sh_next])
            ids_p = plsc.load_gather(ids_v, [c + sh_prev])
            bnd = (ids != ids_n) | m_last
            stt = (ids != ids_p) | m_first
            ps = plsc.cumsum(vals)
            plsc.addupdate_scatter(acc_v, [ids], ps, mask=bnd)
            plsc.addupdate_scatter(acc_v, [ids], vals - ps, mask=stt)

        pltpu.sync_copy(acc_v, out_hbm.at[pl.ds(wid * B, B)])

    return _seg_sum


@functools.lru_cache(maxsize=None)
def _build_j_assemble():
    @functools.partial(
        pl.kernel,
        mesh=plsc.VectorSubcoreMesh(**_SC_MESH),
        compiler_params=_sc_compiler_params(),
        out_type=(
            jax.ShapeDtypeStruct((B,), jnp.float32),
            jax.ShapeDtypeStruct((N * 6,), jnp.float32),
        ),
        scratch_types=[
            pltpu.VMEM((NW * B,), jnp.float32),
            pltpu.VMEM((B,), jnp.float32),
            pltpu.VMEM((B,), jnp.float32),
            pltpu.VMEM((16,), jnp.float32),
            pltpu.VMEM((16,), jnp.float32),
            pltpu.VMEM((16,), jnp.float32),
            pltpu.VMEM((WIN,), jnp.int32),
            pltpu.VMEM((WIN,), jnp.float32),
            pltpu.VMEM((WIN,), jnp.float32),
            pltpu.VMEM((WIN,), jnp.float32),
            pltpu.VMEM((WIN,), jnp.float32),
            pltpu.VMEM((WIN * 6,), jnp.float32),
            pltpu.SemaphoreType.DMA,
        ],
    )
    def _j_assemble(parts_hbm, e0_hbm, batch_hbm, m_hbm, gT_hbm, p_hbm,
                    ec_hbm, j_hbm,
                    parts_v, e0_v, ec_v, px_v, py_v, pz_v, ids_v, m_v,
                    gx_v, gy_v, gz_v, jst_v, sem):
        wid = lax.axis_index("c") * 16 + lax.axis_index("s")

        pltpu.async_copy(parts_hbm, parts_v, sem).wait()
        pltpu.async_copy(e0_hbm, e0_v, sem).wait()
        pltpu.async_copy(p_hbm.at[pl.ds(0, 16)], px_v, sem).wait()
        pltpu.async_copy(p_hbm.at[pl.ds(128, 16)], py_v, sem).wait()
        pltpu.async_copy(p_hbm.at[pl.ds(256, 16)], pz_v, sem).wait()

        @pl.loop(0, B, step=16)
        def _(k):
            a16 = -e0_v[pl.ds(k, 16)]
            for r in range(NW):
                a16 = a16 + parts_v[pl.ds(k + r * B, 16)]
            ec_v[pl.ds(k, 16)] = a16

        pltpu.sync_copy(ec_v.at[pl.ds(wid * 32, 32)],
                        ec_hbm.at[pl.ds(wid * 32, 32)])

        px = px_v[...]
        py = py_v[...]
        pz = pz_v[...]

        iota = lax.iota(jnp.int32, 16)

        for t in range(NWIN):
            gbase = wid * CH + t * WIN
            pltpu.sync_copy(batch_hbm.at[pl.ds(gbase, WIN)], ids_v)
            pltpu.sync_copy(m_hbm.at[pl.ds(gbase, WIN)], m_v)
            pltpu.sync_copy(gT_hbm.at[pl.ds(gbase, WIN)], gx_v)
            pltpu.sync_copy(gT_hbm.at[pl.ds(N + gbase, WIN)], gy_v)
            pltpu.sync_copy(gT_hbm.at[pl.ds(2 * N + gbase, WIN)], gz_v)

            @pl.loop(0, WIN, step=16)
            def _(c):
                ids = ids_v[pl.ds(c, 16)]
                eb = plsc.load_gather(ec_v, [ids])
                mm = m_v[pl.ds(c, 16)]
                a = eb + mm
                idx6 = (c + iota) * 6
                gx = gx_v[pl.ds(c, 16)]
                gy = gy_v[pl.ds(c, 16)]
                gz = gz_v[pl.ds(c, 16)]
                plsc.store_scatter(jst_v, [idx6], gx * eb)
                plsc.store_scatter(jst_v, [idx6 + 1], gy * eb)
                plsc.store_scatter(jst_v, [idx6 + 2], gz * eb)
                plsc.store_scatter(jst_v, [idx6 + 3], a * px)
                plsc.store_scatter(jst_v, [idx6 + 4], a * py)
                plsc.store_scatter(jst_v, [idx6 + 5], a * pz)

            pltpu.sync_copy(jst_v, j_hbm.at[pl.ds(gbase * 6, WIN * 6)])

    return _j_assemble


def kernel(r, v, batch, z, m, emb, W1, b1, W2, b2, E0):
    f32 = jnp.float32
    rT = r.astype(f32).T
    vT = v.astype(f32).T
    batch_i = batch.astype(jnp.int32)
    z3 = z.astype(jnp.int32).reshape(NB, 1, A)
    m3 = m.astype(f32).reshape(NB, 1, A)

    embT = jnp.pad(emb.astype(f32).T, ((0, 0), (0, 128 - NZ)))
    W1eT = W1[3:].astype(f32).T
    W1rT = W1[:3].astype(f32).T
    W1r8 = jnp.pad(W1[:3].astype(f32), ((0, 5), (0, 0)))
    W2pT = jnp.pad(W2.astype(f32).T, ((0, 7), (0, 0)))
    w2c = W2.astype(f32)
    b1c = b1.astype(f32)[:, None]
    b2r = b2.astype(f32).reshape(1, 1)

    ea3, gT, p_out = _tc_stage(rT, vT, z3, m3, embT, W1eT, W1rT, W1r8,
                               W2pT, w2c, b1c, b2r)
    ea_flat = ea3.reshape(N)

    parts = _build_seg_sum()(ea_flat, batch_i)
    Ec, Jf = _build_j_assemble()(parts, E0.astype(f32), batch_i,
                                 m.astype(f32), gT.reshape(3 * N),
                                 p_out.reshape(3 * 128))
    c = jnp.concatenate([Ec, p_out[:, 0]])
    return (c, Jf.reshape(N, 6))
